# Initial kernel scaffold; baseline (speedup 1.0000x reference)
import jax
import jax.numpy as jnp
from jax.experimental import pallas as pl


def kernel(x, edge_index, Win, bin_, Wl1, bl1, Wr1, Wl2, bl2, Wr2, g1, be1, g2, be2, gW, gb, cW1, cb1, cW2, cb2):
    raise NotImplementedError("write your pallas kernel here")


# speedup vs baseline: 3.2447x; 3.2447x over previous
import functools

import jax
import jax.numpy as jnp
from jax import lax
from jax.experimental import pallas as pl
from jax.experimental.pallas import tpu as pltpu
from jax.experimental.pallas import tpu_sc as plsc

N = 10000
E = 320000
D = 128
H = 128
HH = H // 2
C = 40
NB = 3
EPS = 1e-5

NC = 2
NS = 16
NW = NC * NS
K = 128
NCHUNK = 160
NCHUNK_D = NCHUNK // NC
EPAD = NS * NCHUNK * K
NPAD = 10240
RPT = NPAD // NS

_F32 = jnp.float32
_HIGH = lax.Precision.HIGHEST


def _sc_degree_body(dst_hbm, out_hbm, dst_v, ones_v, zrow_v, cacc, sem):
    c = lax.axis_index("c")
    s = lax.axis_index("s")
    for i in range(K // 16):
        ones_v[pl.ds(16 * i, 16)] = jnp.ones((16,), _F32)
    for i in range(RPT // 16):
        zrow_v[pl.ds(16 * i, 16)] = jnp.zeros((16,), _F32)
    pltpu.sync_copy(zrow_v, cacc.at[pl.ds(s * RPT, RPT)])
    pltpu.sync_copy(dst_hbm.at[s, pl.ds(c * NCHUNK_D, NCHUNK_D)], dst_v)
    plsc.subcore_barrier()

    W = 16

    def fire(j, carry):
        pltpu.async_copy(ones_v, cacc.at[dst_v.at[j]], sem, add=True)

        @pl.when(j >= W)
        def _():
            pltpu.make_async_copy(ones_v, cacc.at[dst_v.at[0]], sem).wait()

        return carry

    lax.fori_loop(0, NCHUNK_D, fire, 0)

    def drain(j, carry):
        pltpu.make_async_copy(ones_v, cacc.at[dst_v.at[0]], sem).wait()
        return carry

    lax.fori_loop(0, W, drain, 0)
    plsc.subcore_barrier()
    pltpu.sync_copy(cacc.at[pl.ds(s * RPT, RPT)],
                    out_hbm.at[c, pl.ds(s * RPT, RPT)])


def _sc_aggregate_body(x_hbm, src_hbm, dst_hbm, zeros_hbm, out_hbm,
                       src_v, dst_v, buf0, buf1, acc, sg0, sg1, ss0, ss1):
    c = lax.axis_index("c")
    s = lax.axis_index("s")
    pltpu.sync_copy(zeros_hbm, acc.at[pl.ds(s * RPT, RPT)])
    pltpu.sync_copy(src_hbm.at[c, s], src_v)
    pltpu.sync_copy(dst_hbm.at[s], dst_v)
    plsc.subcore_barrier()

    pltpu.async_copy(x_hbm.at[src_v.at[0]], buf0, sg0)
    NGR = NCHUNK // 2

    def body(g, carry):
        j0 = 2 * g
        pltpu.make_async_copy(x_hbm.at[src_v.at[j0]], buf0, sg0).wait()

        @pl.when(g > 0)
        def _():
            pltpu.make_async_copy(buf1, acc.at[dst_v.at[0]], ss1).wait()

        pltpu.async_copy(x_hbm.at[src_v.at[j0 + 1]], buf1, sg1)
        pltpu.async_copy(buf0, acc.at[dst_v.at[j0]], ss0, add=True)
        pltpu.make_async_copy(x_hbm.at[src_v.at[j0 + 1]], buf1, sg1).wait()
        pltpu.make_async_copy(buf0, acc.at[dst_v.at[0]], ss0).wait()

        @pl.when(g + 1 < NGR)
        def _():
            pltpu.async_copy(x_hbm.at[src_v.at[j0 + 2]], buf0, sg0)

        pltpu.async_copy(buf1, acc.at[dst_v.at[j0 + 1]], ss1, add=True)
        return carry

    lax.fori_loop(0, NGR, body, 0)
    pltpu.make_async_copy(buf1, acc.at[dst_v.at[0]], ss1).wait()
    plsc.subcore_barrier()
    pltpu.sync_copy(acc.at[pl.ds(s * RPT, RPT)],
                    out_hbm.at[c, pl.ds(s * RPT, RPT)])


@functools.cache
def _sc_kernels():
    mesh = plsc.VectorSubcoreMesh(core_axis_name="c", subcore_axis_name="s",
                                  num_cores=NC, num_subcores=NS)
    degree = pl.kernel(
        _sc_degree_body,
        out_type=jax.ShapeDtypeStruct((NC, NPAD), _F32),
        mesh=mesh,
        scratch_types=[
            pltpu.VMEM((NCHUNK_D, K), jnp.int32),
            pltpu.VMEM((K,), _F32),
            pltpu.VMEM((RPT,), _F32),
            pltpu.VMEM_SHARED((NPAD,), _F32),
            pltpu.SemaphoreType.DMA,
        ],
    )
    aggregate = pl.kernel(
        _sc_aggregate_body,
        out_type=jax.ShapeDtypeStruct((NC, NPAD, HH), _F32),
        mesh=mesh,
        compiler_params=pltpu.CompilerParams(use_tc_tiling_on_sc=False),
        scratch_types=[
            pltpu.VMEM((NCHUNK, K), jnp.int32),
            pltpu.VMEM((NCHUNK, K), jnp.int32),
            pltpu.VMEM((K, HH), _F32),
            pltpu.VMEM((K, HH), _F32),
            pltpu.VMEM_SHARED((NPAD, HH), _F32),
            pltpu.SemaphoreType.DMA,
            pltpu.SemaphoreType.DMA,
            pltpu.SemaphoreType.DMA,
            pltpu.SemaphoreType.DMA,
        ],
    )
    return degree, aggregate


def _sc_degree(dstp):
    return _sc_kernels()[0](dstp)


def _sc_aggregate(xf, srcAB, dstp, zeros):
    return _sc_kernels()[1](xf.reshape(2 * N, HH), srcAB, dstp, zeros)


BLKR = 2000
NBLK = N // BLKR


def _matmul(a, b):
    return jnp.dot(a, b, precision=_HIGH, preferred_element_type=_F32)


def _tc_input_proj(x_ref, w_ref, b_ref, o_ref):
    o_ref[...] = _matmul(x_ref[...], w_ref[...]) + b_ref[...]


def _tc_mm_stats(agg_ref, cnt_ref, h_ref, wl_ref, bl_ref, wr_ref,
                 y_ref, st_ref):
    i = pl.program_id(0)
    agg = jnp.concatenate([agg_ref[0], agg_ref[1]], axis=1)
    mean = agg / jnp.maximum(cnt_ref[...], 1.0)
    y = (_matmul(mean, wl_ref[...]) + bl_ref[...]
         + _matmul(h_ref[...], wr_ref[...]))
    y_ref[...] = y
    su = jnp.sum(y, axis=0, keepdims=True)
    sq = jnp.sum(y * y, axis=0, keepdims=True)
    st = jnp.concatenate([su, sq, jnp.zeros((6, H), _F32)], axis=0)

    @pl.when(i == 0)
    def _():
        st_ref[...] = st

    @pl.when(i > 0)
    def _():
        st_ref[...] += st


def _bn(y, st, g, be):
    m = st[0:1] / N
    v = st[1:2] / N - m * m
    return g * (y - m) * lax.rsqrt(v + EPS) + be


def _tc_bn1(y_ref, st_ref, g_ref, be_ref, o_ref):
    o_ref[...] = jnp.maximum(
        _bn(y_ref[...], st_ref[...], g_ref[...], be_ref[...]), 0.0)


def _tc_bn2(y_ref, st_ref, g_ref, be_ref, prev_ref, o_ref):
    bn = jnp.maximum(
        _bn(y_ref[...], st_ref[...], g_ref[...], be_ref[...]), 0.0)
    o_ref[...] = bn + prev_ref[...]


def _tc_bn2_gate(y_ref, st_ref, g_ref, be_ref, gw_ref, gb_ref, prev_ref,
                 o_ref):
    bn = jnp.maximum(
        _bn(y_ref[...], st_ref[...], g_ref[...], be_ref[...]), 0.0)
    prev = prev_ref[...]
    out = bn + prev
    gate = jax.nn.sigmoid(_matmul(prev, gw_ref[...]) + gb_ref[...])
    o_ref[...] = gate * prev + (1.0 - gate) * out


def _tc_head(h_ref, w1_ref, b1_ref, w2_ref, b2_ref, o_ref):
    hid = jnp.maximum(_matmul(h_ref[...], w1_ref[...]) + b1_ref[...], 0.0)
    o_ref[...] = _matmul(hid, w2_ref[...]) + b2_ref[...]


def _call(body, out_shape, *args):
    return pl.pallas_call(
        body, out_shape=jax.ShapeDtypeStruct(out_shape, _F32))(*args)


def _rows_spec(shape2):
    return pl.BlockSpec((BLKR,) + shape2[1:], lambda i: (i,) + (0,) * (len(shape2) - 1))


def _full_spec(shape):
    return pl.BlockSpec(shape, lambda i: (0,) * len(shape))


def _mm_stats(aggP, cnt_col, h, wlT, bl, wrT):
    return pl.pallas_call(
        _tc_mm_stats,
        grid=(NBLK,),
        in_specs=[
            pl.BlockSpec((NC, BLKR, HH), lambda i: (0, i, 0)),
            _rows_spec((N, 1)),
            _rows_spec((N, H)),
            _full_spec((H, H)),
            _full_spec((1, H)),
            _full_spec((H, H)),
        ],
        out_specs=[_rows_spec((N, H)), _full_spec((8, H))],
        out_shape=[jax.ShapeDtypeStruct((N, H), _F32),
                   jax.ShapeDtypeStruct((8, H), _F32)],
    )(aggP, cnt_col, h, wlT, bl, wrT)


def _bn_apply(body, ins_full, y, st, *blocked):
    in_specs = ([_rows_spec((N, H)), _full_spec((8, H))]
                + [_full_spec(a.shape) for a in ins_full]
                + [_rows_spec((N, H)) for _ in blocked])
    return pl.pallas_call(
        body,
        grid=(NBLK,),
        in_specs=in_specs,
        out_specs=_rows_spec((N, H)),
        out_shape=jax.ShapeDtypeStruct((N, H), _F32),
    )(y, st, *ins_full, *blocked)


def kernel(x, edge_index, Win, bin_, Wl1, bl1, Wr1, Wl2, bl2, Wr2, g1, be1,
           g2, be2, gW, gb, cW1, cb1, cW2, cb2):
    src = edge_index[0]
    dst = edge_index[1]
    pad = EPAD - E
    base = jnp.concatenate([src, jnp.zeros((pad,), jnp.int32)]
                           ).reshape(NS, NCHUNK, K)
    srcAB = jnp.stack([2 * base, 2 * base + 1])
    dstp = jnp.concatenate([dst, jnp.full((pad,), NPAD - 1, jnp.int32)]
                           ).reshape(NS, NCHUNK, K)
    zeros = jnp.zeros((RPT, HH), _F32)

    cntP = _sc_degree(dstp)
    cnt_col = (cntP[0] + cntP[1])[:N, None]

    row = lambda b: b.reshape(1, -1)
    prev = _call(_tc_input_proj, (N, H), x, Win.T, row(bin_))
    for i in range(NB):
        aggP = _sc_aggregate(prev, srcAB, dstp, zeros)
        y, st = _mm_stats(aggP, cnt_col, prev, Wl1[i].T, row(bl1[i]), Wr1[i].T)
        h = _bn_apply(_tc_bn1, [row(g1[i]), row(be1[i])], y, st)
        aggP = _sc_aggregate(h, srcAB, dstp, zeros)
        y, st = _mm_stats(aggP, cnt_col, h, Wl2[i].T, row(bl2[i]), Wr2[i].T)
        if i == 0:
            prev = _bn_apply(_tc_bn2, [row(g2[i]), row(be2[i])], y, st, prev)
        else:
            prev = _bn_apply(_tc_bn2_gate,
                             [row(g2[i]), row(be2[i]),
                              gW[i - 1].T, row(gb[i - 1])], y, st, prev)
    return _call(_tc_head, (N, C), prev, cW1.T, row(cb1), cW2.T, row(cb2))


# speedup vs baseline: 3.4682x; 1.0689x over previous
import functools

import jax
import jax.numpy as jnp
from jax import lax
from jax.experimental import pallas as pl
from jax.experimental.pallas import tpu as pltpu
from jax.experimental.pallas import tpu_sc as plsc

N = 10000
E = 320000
D = 128
H = 128
HH = H // 2
C = 40
NB = 3
EPS = 1e-5

NC = 2
NS = 16
NW = NC * NS
K = 128
NCHUNK = 160
NCHUNK_D = NCHUNK // NC
EPAD = NS * NCHUNK * K
NPAD = 10240
RPT = NPAD // NS

_F32 = jnp.float32
_HIGH = lax.Precision.HIGHEST


def _sc_degree_body(dst_hbm, out_hbm, dst_v, ones_v, zrow_v, cacc, sem):
    c = lax.axis_index("c")
    s = lax.axis_index("s")
    for i in range(K // 16):
        ones_v[pl.ds(16 * i, 16)] = jnp.ones((16,), _F32)
    for i in range(RPT // 16):
        zrow_v[pl.ds(16 * i, 16)] = jnp.zeros((16,), _F32)
    pltpu.sync_copy(zrow_v, cacc.at[pl.ds(s * RPT, RPT)])
    pltpu.sync_copy(dst_hbm.at[s, pl.ds(c * NCHUNK_D, NCHUNK_D)], dst_v)
    plsc.subcore_barrier()

    W = 16

    def fire(j, carry):
        pltpu.async_copy(ones_v, cacc.at[dst_v.at[j]], sem, add=True)

        @pl.when(j >= W)
        def _():
            pltpu.make_async_copy(ones_v, cacc.at[dst_v.at[0]], sem).wait()

        return carry

    lax.fori_loop(0, NCHUNK_D, fire, 0)

    def drain(j, carry):
        pltpu.make_async_copy(ones_v, cacc.at[dst_v.at[0]], sem).wait()
        return carry

    lax.fori_loop(0, W, drain, 0)
    plsc.subcore_barrier()
    pltpu.sync_copy(cacc.at[pl.ds(s * RPT, RPT)],
                    out_hbm.at[c, pl.ds(s * RPT, RPT)])


NBUF = 4
NR = NCHUNK // NBUF


def _sc_aggregate_body(x_hbm, src_hbm, dst_hbm, zeros_hbm, out_hbm,
                       src_v, dst_v, bufs, acc, sgs, sss):
    c = lax.axis_index("c")
    s = lax.axis_index("s")
    pltpu.sync_copy(zeros_hbm, acc.at[pl.ds(s * RPT, RPT)])
    pltpu.sync_copy(src_hbm.at[c, s], src_v)
    pltpu.sync_copy(dst_hbm.at[s], dst_v)
    plsc.subcore_barrier()

    pltpu.async_copy(x_hbm.at[src_v.at[0]], bufs[0], sgs[0])
    pltpu.async_copy(x_hbm.at[src_v.at[1]], bufs[1], sgs[1])

    def rnd(r, carry):
        for slot in range(NBUF):
            v = NBUF * r + slot
            bc, bn = slot, (slot + 2) % NBUF
            pltpu.make_async_copy(x_hbm.at[src_v.at[v]], bufs[bc],
                                  sgs[bc]).wait()
            pltpu.async_copy(bufs[bc], acc.at[dst_v.at[v]], sss[bc],
                             add=True)
            if slot < 2:
                @pl.when(r > 0)
                def _(bn=bn):
                    pltpu.make_async_copy(bufs[bn], acc.at[dst_v.at[0]],
                                          sss[bn]).wait()
                pltpu.async_copy(x_hbm.at[src_v.at[v + 2]], bufs[bn],
                                 sgs[bn])
            else:
                @pl.when(r < NR - 1)
                def _(v=v, bn=bn):
                    pltpu.make_async_copy(bufs[bn], acc.at[dst_v.at[0]],
                                          sss[bn]).wait()
                    pltpu.async_copy(x_hbm.at[src_v.at[v + 2]], bufs[bn],
                                     sgs[bn])
        return carry

    lax.fori_loop(0, NR, rnd, 0)
    for b in range(NBUF):
        pltpu.make_async_copy(bufs[b], acc.at[dst_v.at[0]], sss[b]).wait()
    plsc.subcore_barrier()
    pltpu.sync_copy(acc.at[pl.ds(s * RPT, RPT)],
                    out_hbm.at[c, pl.ds(s * RPT, RPT)])


@functools.cache
def _sc_kernels():
    mesh = plsc.VectorSubcoreMesh(core_axis_name="c", subcore_axis_name="s",
                                  num_cores=NC, num_subcores=NS)
    degree = pl.kernel(
        _sc_degree_body,
        out_type=jax.ShapeDtypeStruct((NC, NPAD), _F32),
        mesh=mesh,
        scratch_types=[
            pltpu.VMEM((NCHUNK_D, K), jnp.int32),
            pltpu.VMEM((K,), _F32),
            pltpu.VMEM((RPT,), _F32),
            pltpu.VMEM_SHARED((NPAD,), _F32),
            pltpu.SemaphoreType.DMA,
        ],
    )
    aggregate = pl.kernel(
        _sc_aggregate_body,
        out_type=jax.ShapeDtypeStruct((NC, NPAD, HH), _F32),
        mesh=mesh,
        compiler_params=pltpu.CompilerParams(use_tc_tiling_on_sc=False),
        scratch_types=[
            pltpu.VMEM((NCHUNK, K), jnp.int32),
            pltpu.VMEM((NCHUNK, K), jnp.int32),
            [pltpu.VMEM((K, HH), _F32) for _ in range(NBUF)],
            pltpu.VMEM_SHARED((NPAD, HH), _F32),
            [pltpu.SemaphoreType.DMA for _ in range(NBUF)],
            [pltpu.SemaphoreType.DMA for _ in range(NBUF)],
        ],
    )
    return degree, aggregate


def _sc_degree(dstp):
    return _sc_kernels()[0](dstp)


def _sc_aggregate(xf, srcAB, dstp, zeros):
    return _sc_kernels()[1](xf.reshape(2 * N, HH), srcAB, dstp, zeros)


BLKR = 2000
NBLK = N // BLKR


def _matmul(a, b):
    return jnp.dot(a, b, precision=_HIGH, preferred_element_type=_F32)


def _tc_input_proj(x_ref, w_ref, b_ref, o_ref):
    o_ref[...] = _matmul(x_ref[...], w_ref[...]) + b_ref[...]


def _tc_mm_stats(agg_ref, cnt_ref, h_ref, wl_ref, bl_ref, wr_ref,
                 y_ref, st_ref):
    i = pl.program_id(0)
    agg = jnp.concatenate([agg_ref[0], agg_ref[1]], axis=1)
    mean = agg / jnp.maximum(cnt_ref[...], 1.0)
    y = (_matmul(mean, wl_ref[...]) + bl_ref[...]
         + _matmul(h_ref[...], wr_ref[...]))
    y_ref[...] = y
    su = jnp.sum(y, axis=0, keepdims=True)
    sq = jnp.sum(y * y, axis=0, keepdims=True)
    st = jnp.concatenate([su, sq, jnp.zeros((6, H), _F32)], axis=0)

    @pl.when(i == 0)
    def _():
        st_ref[...] = st

    @pl.when(i > 0)
    def _():
        st_ref[...] += st


def _bn(y, st, g, be):
    m = st[0:1] / N
    v = st[1:2] / N - m * m
    return g * (y - m) * lax.rsqrt(v + EPS) + be


def _tc_bn1(y_ref, st_ref, g_ref, be_ref, o_ref):
    o_ref[...] = jnp.maximum(
        _bn(y_ref[...], st_ref[...], g_ref[...], be_ref[...]), 0.0)


def _tc_bn2(y_ref, st_ref, g_ref, be_ref, prev_ref, o_ref):
    bn = jnp.maximum(
        _bn(y_ref[...], st_ref[...], g_ref[...], be_ref[...]), 0.0)
    o_ref[...] = bn + prev_ref[...]


def _tc_bn2_gate(y_ref, st_ref, g_ref, be_ref, gw_ref, gb_ref, prev_ref,
                 o_ref):
    bn = jnp.maximum(
        _bn(y_ref[...], st_ref[...], g_ref[...], be_ref[...]), 0.0)
    prev = prev_ref[...]
    out = bn + prev
    gate = jax.nn.sigmoid(_matmul(prev, gw_ref[...]) + gb_ref[...])
    o_ref[...] = gate * prev + (1.0 - gate) * out


def _tc_head(h_ref, w1_ref, b1_ref, w2_ref, b2_ref, o_ref):
    hid = jnp.maximum(_matmul(h_ref[...], w1_ref[...]) + b1_ref[...], 0.0)
    o_ref[...] = _matmul(hid, w2_ref[...]) + b2_ref[...]


def _call(body, out_shape, *args):
    return pl.pallas_call(
        body, out_shape=jax.ShapeDtypeStruct(out_shape, _F32))(*args)


def _rows_spec(shape2):
    return pl.BlockSpec((BLKR,) + shape2[1:], lambda i: (i,) + (0,) * (len(shape2) - 1))


def _full_spec(shape):
    return pl.BlockSpec(shape, lambda i: (0,) * len(shape))


def _mm_stats(aggP, cnt_col, h, wlT, bl, wrT):
    return pl.pallas_call(
        _tc_mm_stats,
        grid=(NBLK,),
        in_specs=[
            pl.BlockSpec((NC, BLKR, HH), lambda i: (0, i, 0)),
            _rows_spec((N, 1)),
            _rows_spec((N, H)),
            _full_spec((H, H)),
            _full_spec((1, H)),
            _full_spec((H, H)),
        ],
        out_specs=[_rows_spec((N, H)), _full_spec((8, H))],
        out_shape=[jax.ShapeDtypeStruct((N, H), _F32),
                   jax.ShapeDtypeStruct((8, H), _F32)],
    )(aggP, cnt_col, h, wlT, bl, wrT)


def _bn_apply(body, ins_full, y, st, *blocked):
    in_specs = ([_rows_spec((N, H)), _full_spec((8, H))]
                + [_full_spec(a.shape) for a in ins_full]
                + [_rows_spec((N, H)) for _ in blocked])
    return pl.pallas_call(
        body,
        grid=(NBLK,),
        in_specs=in_specs,
        out_specs=_rows_spec((N, H)),
        out_shape=jax.ShapeDtypeStruct((N, H), _F32),
    )(y, st, *ins_full, *blocked)


def kernel(x, edge_index, Win, bin_, Wl1, bl1, Wr1, Wl2, bl2, Wr2, g1, be1,
           g2, be2, gW, gb, cW1, cb1, cW2, cb2):
    src = edge_index[0]
    dst = edge_index[1]
    pad = EPAD - E
    base = jnp.concatenate([src, jnp.zeros((pad,), jnp.int32)]
                           ).reshape(NS, NCHUNK, K)
    srcAB = jnp.stack([2 * base, 2 * base + 1])
    dstp = jnp.concatenate([dst, jnp.full((pad,), NPAD - 1, jnp.int32)]
                           ).reshape(NS, NCHUNK, K)
    zeros = jnp.zeros((RPT, HH), _F32)

    cntP = _sc_degree(dstp)
    cnt_col = (cntP[0] + cntP[1])[:N, None]

    row = lambda b: b.reshape(1, -1)
    prev = _call(_tc_input_proj, (N, H), x, Win.T, row(bin_))
    for i in range(NB):
        aggP = _sc_aggregate(prev, srcAB, dstp, zeros)
        y, st = _mm_stats(aggP, cnt_col, prev, Wl1[i].T, row(bl1[i]), Wr1[i].T)
        h = _bn_apply(_tc_bn1, [row(g1[i]), row(be1[i])], y, st)
        aggP = _sc_aggregate(h, srcAB, dstp, zeros)
        y, st = _mm_stats(aggP, cnt_col, h, Wl2[i].T, row(bl2[i]), Wr2[i].T)
        if i == 0:
            prev = _bn_apply(_tc_bn2, [row(g2[i]), row(be2[i])], y, st, prev)
        else:
            prev = _bn_apply(_tc_bn2_gate,
                             [row(g2[i]), row(be2[i]),
                              gW[i - 1].T, row(gb[i - 1])], y, st, prev)
    return _call(_tc_head, (N, C), prev, cW1.T, row(cb1), cW2.T, row(cb2))


# speedup vs baseline: 8.5183x; 2.4561x over previous
import functools

import jax
import jax.numpy as jnp
from jax import lax
from jax.experimental import pallas as pl
from jax.experimental.pallas import tpu as pltpu
from jax.experimental.pallas import tpu_sc as plsc

N = 10000
E = 320000
D = 128
H = 128
HH = H // 2
C = 40
NB = 3
EPS = 1e-5

NC = 2
NS = 16
NW = NC * NS
K = 128
NCHUNK = 160
NCHUNK_D = NCHUNK // NC
EPAD = NS * NCHUNK * K
NPAD = 10240
RPT = NPAD // NS

_F32 = jnp.float32
_HIGH = lax.Precision.HIGHEST


def _sc_degree_body(dst_hbm, out_hbm, dst_v, ones_v, zrow_v, cacc, sem):
    c = lax.axis_index("c")
    s = lax.axis_index("s")
    for i in range(K // 16):
        ones_v[pl.ds(16 * i, 16)] = jnp.ones((16,), _F32)
    for i in range(RPT // 16):
        zrow_v[pl.ds(16 * i, 16)] = jnp.zeros((16,), _F32)
    pltpu.sync_copy(zrow_v, cacc.at[pl.ds(s * RPT, RPT)])
    pltpu.sync_copy(dst_hbm.at[s, pl.ds(c * NCHUNK_D, NCHUNK_D)], dst_v)
    plsc.subcore_barrier()

    W = 16

    def fire(j, carry):
        pltpu.async_copy(ones_v, cacc.at[dst_v.at[j]], sem, add=True)

        @pl.when(j >= W)
        def _():
            pltpu.make_async_copy(ones_v, cacc.at[dst_v.at[0]], sem).wait()

        return carry

    lax.fori_loop(0, NCHUNK_D, fire, 0)

    def drain(j, carry):
        pltpu.make_async_copy(ones_v, cacc.at[dst_v.at[0]], sem).wait()
        return carry

    lax.fori_loop(0, W, drain, 0)
    plsc.subcore_barrier()
    pltpu.sync_copy(cacc.at[pl.ds(s * RPT, RPT)],
                    out_hbm.at[c, pl.ds(s * RPT, RPT)])


NBUF = 4
IDXR = 8
NR8 = NCHUNK // IDXR
NPA = 10016
RPA = NPA // NS
TROWS = N // NS


def _sc_aggregate_body(xs_hbm, idx_hbm, zeros_hbm, out_hbm,
                       bufs, idxb, table, acc, sgs, sss, sis):
    c = lax.axis_index("c")
    sid = lax.axis_index("s")
    pltpu.sync_copy(xs_hbm.at[c, pl.ds(sid * TROWS, TROWS)],
                    table.at[pl.ds(sid * TROWS, TROWS)])
    pltpu.sync_copy(zeros_hbm, acc.at[pl.ds(sid * RPA, RPA)])
    for w in range(6):
        pltpu.async_copy(idx_hbm.at[sid, w], idxb[w], sis[w])
    plsc.subcore_barrier()
    pltpu.make_async_copy(idx_hbm.at[sid, 0], idxb[0], sis[0]).wait()
    pltpu.async_copy(table.at[idxb[0].at[0]], bufs[0], sgs[0])
    pltpu.make_async_copy(idx_hbm.at[sid, 0], idxb[1], sis[1]).wait()
    pltpu.async_copy(table.at[idxb[1].at[0]], bufs[1], sgs[1])

    def rnd(r, carry):
        for t in range(IDXR):
            v = IDXR * r + t
            b4, b4n, b8, b8n = t % 4, (t + 2) % 4, t, (t + 2) % IDXR

            def wait_ss(b=b4n):
                pltpu.make_async_copy(bufs[b], acc.at[idxb[0].at[1]],
                                      sss[b]).wait()

            def stage_idx(v=v, b=(t + 6) % IDXR):
                pltpu.async_copy(idx_hbm.at[sid, v + 6], idxb[b], sis[b])

            def gather_next(b8n=b8n, b4n=b4n):
                pltpu.make_async_copy(idx_hbm.at[sid, 0], idxb[b8n],
                                      sis[b8n]).wait()
                pltpu.async_copy(table.at[idxb[b8n].at[0]], bufs[b4n],
                                 sgs[b4n])

            pltpu.make_async_copy(table.at[idxb[b8].at[0]], bufs[b4],
                                  sgs[b4]).wait()
            pltpu.async_copy(bufs[b4], acc.at[idxb[b8].at[1]], sss[b4],
                             add=True)
            if t < 2:
                pl.when(r > 0)(wait_ss)
                stage_idx()
                gather_next()
            else:
                wait_ss()
                pl.when(r < NR8 - 1)(stage_idx)
                if t < 6:
                    gather_next()
                else:
                    pl.when(r < NR8 - 1)(gather_next)
        return carry

    lax.fori_loop(0, NR8, rnd, 0)
    for b in (2, 3):
        pltpu.make_async_copy(bufs[b], acc.at[idxb[0].at[1]], sss[b]).wait()
    plsc.subcore_barrier()
    pltpu.sync_copy(acc.at[pl.ds(sid * RPA, RPA)],
                    out_hbm.at[c, pl.ds(sid * RPA, RPA)])


@functools.cache
def _sc_kernels():
    mesh = plsc.VectorSubcoreMesh(core_axis_name="c", subcore_axis_name="s",
                                  num_cores=NC, num_subcores=NS)
    degree = pl.kernel(
        _sc_degree_body,
        out_type=jax.ShapeDtypeStruct((NC, NPAD), _F32),
        mesh=mesh,
        scratch_types=[
            pltpu.VMEM((NCHUNK_D, K), jnp.int32),
            pltpu.VMEM((K,), _F32),
            pltpu.VMEM((RPT,), _F32),
            pltpu.VMEM_SHARED((NPAD,), _F32),
            pltpu.SemaphoreType.DMA,
        ],
    )
    aggregate = pl.kernel(
        _sc_aggregate_body,
        out_type=jax.ShapeDtypeStruct((NC, NPA, HH), _F32),
        mesh=mesh,
        compiler_params=pltpu.CompilerParams(use_tc_tiling_on_sc=False),
        scratch_types=[
            [pltpu.VMEM((K, HH), _F32) for _ in range(NBUF)],
            [pltpu.VMEM((2, K), jnp.int32) for _ in range(IDXR)],
            pltpu.VMEM_SHARED((N, HH), _F32),
            pltpu.VMEM_SHARED((NPA, HH), _F32),
            [pltpu.SemaphoreType.DMA for _ in range(NBUF)],
            [pltpu.SemaphoreType.DMA for _ in range(NBUF)],
            [pltpu.SemaphoreType.DMA for _ in range(IDXR)],
        ],
    )
    return degree, aggregate


def _sc_degree(dstp):
    return _sc_kernels()[0](dstp)


def _sc_aggregate(xs, idxp, zeros):
    return _sc_kernels()[1](xs, idxp, zeros)


BLKR = 2000
NBLK = N // BLKR


def _matmul(a, b):
    return jnp.dot(a, b, precision=_HIGH, preferred_element_type=_F32)


def _tc_input_proj(x_ref, w_ref, b_ref, o_ref, o2_ref):
    _split_store(o_ref, o2_ref, _matmul(x_ref[...], w_ref[...]) + b_ref[...])


def _tc_mm_stats(agg_ref, cnt_ref, h_ref, wl_ref, bl_ref, wr_ref,
                 y_ref, st_ref):
    i = pl.program_id(0)
    agg = jnp.concatenate([agg_ref[0], agg_ref[1]], axis=1)
    mean = agg / jnp.maximum(cnt_ref[...], 1.0)
    y = (_matmul(mean, wl_ref[...]) + bl_ref[...]
         + _matmul(h_ref[...], wr_ref[...]))
    y_ref[...] = y
    su = jnp.sum(y, axis=0, keepdims=True)
    sq = jnp.sum(y * y, axis=0, keepdims=True)
    st = jnp.concatenate([su, sq, jnp.zeros((6, H), _F32)], axis=0)

    @pl.when(i == 0)
    def _():
        st_ref[...] = st

    @pl.when(i > 0)
    def _():
        st_ref[...] += st


def _bn(y, st, g, be):
    m = st[0:1] / N
    v = st[1:2] / N - m * m
    return g * (y - m) * lax.rsqrt(v + EPS) + be


def _split_store(o_ref, o2_ref, val):
    o_ref[...] = val
    o2_ref[0] = val[:, :HH]
    o2_ref[1] = val[:, HH:]


def _tc_bn1(y_ref, st_ref, g_ref, be_ref, o_ref, o2_ref):
    _split_store(o_ref, o2_ref, jnp.maximum(
        _bn(y_ref[...], st_ref[...], g_ref[...], be_ref[...]), 0.0))


def _tc_bn2(y_ref, st_ref, g_ref, be_ref, prev_ref, o_ref, o2_ref):
    bn = jnp.maximum(
        _bn(y_ref[...], st_ref[...], g_ref[...], be_ref[...]), 0.0)
    _split_store(o_ref, o2_ref, bn + prev_ref[...])


def _tc_bn2_gate(y_ref, st_ref, g_ref, be_ref, gw_ref, gb_ref, prev_ref,
                 o_ref, o2_ref):
    bn = jnp.maximum(
        _bn(y_ref[...], st_ref[...], g_ref[...], be_ref[...]), 0.0)
    prev = prev_ref[...]
    out = bn + prev
    gate = jax.nn.sigmoid(_matmul(prev, gw_ref[...]) + gb_ref[...])
    _split_store(o_ref, o2_ref, gate * prev + (1.0 - gate) * out)


def _tc_head(h_ref, w1_ref, b1_ref, w2_ref, b2_ref, o_ref):
    hid = jnp.maximum(_matmul(h_ref[...], w1_ref[...]) + b1_ref[...], 0.0)
    o_ref[...] = _matmul(hid, w2_ref[...]) + b2_ref[...]


def _call(body, out_shape, *args):
    return pl.pallas_call(
        body, out_shape=jax.ShapeDtypeStruct(out_shape, _F32))(*args)


def _rows_spec(shape2):
    return pl.BlockSpec((BLKR,) + shape2[1:], lambda i: (i,) + (0,) * (len(shape2) - 1))


def _full_spec(shape):
    return pl.BlockSpec(shape, lambda i: (0,) * len(shape))


def _mm_stats(aggP, cnt_col, h, wlT, bl, wrT):
    return pl.pallas_call(
        _tc_mm_stats,
        grid=(NBLK,),
        in_specs=[
            pl.BlockSpec((NC, BLKR, HH), lambda i: (0, i, 0)),
            _rows_spec((N, 1)),
            _rows_spec((N, H)),
            _full_spec((H, H)),
            _full_spec((1, H)),
            _full_spec((H, H)),
        ],
        out_specs=[_rows_spec((N, H)), _full_spec((8, H))],
        out_shape=[jax.ShapeDtypeStruct((N, H), _F32),
                   jax.ShapeDtypeStruct((8, H), _F32)],
    )(aggP, cnt_col, h, wlT, bl, wrT)


_SPLIT_SPEC = pl.BlockSpec((NC, BLKR, HH), lambda i: (0, i, 0))


def _bn_apply(body, ins_full, y, st, *blocked):
    in_specs = ([_rows_spec((N, H)), _full_spec((8, H))]
                + [_full_spec(a.shape) for a in ins_full]
                + [_rows_spec((N, H)) for _ in blocked])
    return pl.pallas_call(
        body,
        grid=(NBLK,),
        in_specs=in_specs,
        out_specs=[_rows_spec((N, H)), _SPLIT_SPEC],
        out_shape=[jax.ShapeDtypeStruct((N, H), _F32),
                   jax.ShapeDtypeStruct((NC, N, HH), _F32)],
    )(y, st, *ins_full, *blocked)


def _input_proj(x, wT, b):
    return pl.pallas_call(
        _tc_input_proj,
        out_shape=[jax.ShapeDtypeStruct((N, H), _F32),
                   jax.ShapeDtypeStruct((NC, N, HH), _F32)],
    )(x, wT, b)


def kernel(x, edge_index, Win, bin_, Wl1, bl1, Wr1, Wl2, bl2, Wr2, g1, be1,
           g2, be2, gW, gb, cW1, cb1, cW2, cb2):
    src = edge_index[0]
    dst = edge_index[1]
    pad = EPAD - E
    srcp = jnp.concatenate([src, jnp.zeros((pad,), jnp.int32)]
                           ).reshape(NS, NCHUNK, K)
    dstp = jnp.concatenate([dst, jnp.full((pad,), NPA - 1, jnp.int32)]
                           ).reshape(NS, NCHUNK, K)
    idxp = jnp.stack([srcp, dstp], axis=2)
    zeros = jnp.zeros((RPA, HH), _F32)

    cntP = _sc_degree(dstp)
    cnt_col = (cntP[0] + cntP[1])[:N, None]

    row = lambda b: b.reshape(1, -1)
    prev, prev_s = _input_proj(x, Win.T, row(bin_))
    for i in range(NB):
        aggP = _sc_aggregate(prev_s, idxp, zeros)
        y, st = _mm_stats(aggP, cnt_col, prev, Wl1[i].T, row(bl1[i]), Wr1[i].T)
        h, h_s = _bn_apply(_tc_bn1, [row(g1[i]), row(be1[i])], y, st)
        aggP = _sc_aggregate(h_s, idxp, zeros)
        y, st = _mm_stats(aggP, cnt_col, h, Wl2[i].T, row(bl2[i]), Wr2[i].T)
        if i == 0:
            prev, prev_s = _bn_apply(_tc_bn2, [row(g2[i]), row(be2[i])],
                                     y, st, prev)
        else:
            prev, prev_s = _bn_apply(_tc_bn2_gate,
                                     [row(g2[i]), row(be2[i]),
                                      gW[i - 1].T, row(gb[i - 1])], y, st,
                                     prev)
    return _call(_tc_head, (N, C), prev, cW1.T, row(cb1), cW2.T, row(cb2))
